# baseline (speedup 1.0000x reference)
import jax
import jax.numpy as jnp
from jax.experimental import pallas as pl
from jax.experimental.pallas import tpu as pltpu

NUM_CODES = 1024
DIM = 64
COMMIT = 0.25
ROW_BLOCK = 4096


def _vq_kernel(x_ref, w_ref, q_ref, loss_ref):
    x = x_ref[...]
    w = w_ref[...]
    bb, _, pix = x.shape
    xt = x.transpose(0, 2, 1).reshape(bb * pix, DIM)

    wsq = jnp.sum(w * w, axis=1)[None, :]
    xsq = jnp.sum(xt * xt, axis=1, keepdims=True)
    xw = jax.lax.dot_general(
        xt, w, (((1,), (1,)), ((), ())),
        preferred_element_type=jnp.float32)
    d2 = (xsq - 2.0 * xw) + wsq
    scores = jnp.sqrt(jnp.maximum(d2, 0.0))

    m = jnp.min(scores, axis=1, keepdims=True)
    iota = jax.lax.broadcasted_iota(jnp.int32, scores.shape, 1)
    masked = jnp.where(scores <= m, iota, NUM_CODES)
    idx = jnp.min(masked, axis=1, keepdims=True)

    onehot = (iota == idx).astype(jnp.float32)
    q = jax.lax.dot_general(
        onehot, w, (((1,), (0,)), ((), ())),
        preferred_element_type=jnp.float32)
    q_ref[...] = q.reshape(bb, pix, DIM).transpose(0, 2, 1)

    loss_ref[...] = jnp.full((1, 8, 128), jnp.sum(m * m), jnp.float32)


@jax.jit
def kernel(inputs, weight):
    B, C, H, W = inputs.shape
    rows = B * H * W
    bb = ROW_BLOCK // (H * W)
    x = inputs.reshape(B, C, H * W)

    q, loss = pl.pallas_call(
        _vq_kernel,
        grid=(rows // ROW_BLOCK,),
        in_specs=[
            pl.BlockSpec((bb, C, H * W), lambda b: (b, 0, 0)),
            pl.BlockSpec((NUM_CODES, DIM), lambda b: (0, 0)),
        ],
        out_specs=[
            pl.BlockSpec((bb, C, H * W), lambda b: (b, 0, 0)),
            pl.BlockSpec((1, 8, 128), lambda b: (b, 0, 0)),
        ],
        out_shape=[
            jax.ShapeDtypeStruct((B, C, H * W), jnp.float32),
            jax.ShapeDtypeStruct((rows // ROW_BLOCK, 8, 128), jnp.float32),
        ],
        compiler_params=pltpu.CompilerParams(
            dimension_semantics=("parallel",),
        ),
    )(x, weight)

    c_loss = (1.0 + COMMIT) * jnp.sum(loss[:, 0, 0]) / (B * C * H * W)
    return c_loss, q.reshape(B, C, H, W)


# speedup vs baseline: 1.2625x; 1.2625x over previous
import jax
import jax.numpy as jnp
from jax.experimental import pallas as pl
from jax.experimental.pallas import tpu as pltpu

NUM_CODES = 1024
DIM = 64
COMMIT = 0.25
ROW_BLOCK = 8192


def _vq_kernel(xt_ref, w_ref, q_ref, loss_ref):
    xt = xt_ref[...]
    w = w_ref[...]

    wsq = jnp.sum(w * w, axis=1)[None, :]
    xsq = jnp.sum(xt * xt, axis=1, keepdims=True)
    xw = jax.lax.dot_general(
        xt, w, (((1,), (1,)), ((), ())),
        preferred_element_type=jnp.float32)
    d2 = (xsq - 2.0 * xw) + wsq
    scores = jnp.sqrt(jnp.maximum(d2, 0.0))

    m = jnp.min(scores, axis=1, keepdims=True)
    iota = jax.lax.broadcasted_iota(jnp.int32, scores.shape, 1)
    masked = jnp.where(scores <= m, iota, NUM_CODES)
    idx = jnp.min(masked, axis=1, keepdims=True)

    onehot = (iota == idx).astype(jnp.float32)
    q = jax.lax.dot_general(
        onehot, w, (((1,), (0,)), ((), ())),
        preferred_element_type=jnp.float32)
    q_ref[...] = q

    loss_ref[...] = jnp.full((1, 8, 128), jnp.sum(m * m), jnp.float32)


@jax.jit
def kernel(inputs, weight):
    B, C, H, W = inputs.shape
    rows = B * H * W
    xt = inputs.transpose(0, 2, 3, 1).reshape(rows, C)

    q, loss = pl.pallas_call(
        _vq_kernel,
        grid=(rows // ROW_BLOCK,),
        in_specs=[
            pl.BlockSpec((ROW_BLOCK, DIM), lambda b: (b, 0)),
            pl.BlockSpec((NUM_CODES, DIM), lambda b: (0, 0)),
        ],
        out_specs=[
            pl.BlockSpec((ROW_BLOCK, DIM), lambda b: (b, 0)),
            pl.BlockSpec((1, 8, 128), lambda b: (b, 0, 0)),
        ],
        out_shape=[
            jax.ShapeDtypeStruct((rows, DIM), jnp.float32),
            jax.ShapeDtypeStruct((rows // ROW_BLOCK, 8, 128), jnp.float32),
        ],
        compiler_params=pltpu.CompilerParams(
            dimension_semantics=("parallel",),
        ),
    )(xt, weight)

    c_loss = (1.0 + COMMIT) * jnp.sum(loss[:, 0, 0]) / (B * C * H * W)
    return c_loss, q.reshape(B, H, W, C).transpose(0, 3, 1, 2)
